# baseline (speedup 1.0000x reference)
import functools

import jax
import jax.numpy as jnp
from jax import lax
from jax.experimental import pallas as pl
from jax.experimental.pallas import tpu as pltpu
from jax.experimental.pallas import tpu_sc as plsc

B = 16384
D = 64
_L = 16

_info = plsc.get_sparse_core_info()
_NC, _NS = _info.num_cores, _info.num_subcores
_NW = _NC * _NS
_W = B // _NW
_GROUPS = _W // _L


def _make_kernel():
    mesh = plsc.VectorSubcoreMesh(core_axis_name="c", subcore_axis_name="s")

    @functools.partial(
        pl.kernel,
        mesh=mesh,
        out_type=[
            jax.ShapeDtypeStruct((B,), jnp.float32),
            jax.ShapeDtypeStruct((D, B), jnp.float32),
            jax.ShapeDtypeStruct((D, B), jnp.float32),
        ],
        scratch_types=[
            pltpu.VMEM((D, _W // 2), jnp.float32),
            pltpu.VMEM((D, _W // 2), jnp.float32),
            pltpu.VMEM((D, _W // 2), jnp.float32),
            pltpu.VMEM((D, _W // 2), jnp.float32),
            pltpu.VMEM((_W,), jnp.float32),
            pltpu.SemaphoreType.DMA,
            pltpu.SemaphoreType.DMA,
        ],
        compiler_params=pltpu.CompilerParams(needs_layout_passes=False),
    )
    def dot_kernel(gut_hbm, git_hbm, xui_hbm, gout_u, gout_i,
                   u0_v, i0_v, u1_v, i1_v, o_v, sem_in, sem_out):
        wid = lax.axis_index("s") * _NC + lax.axis_index("c")
        base = wid * _W
        half = _W // 2
        bufs = [(u0_v, i0_v), (u1_v, i1_v)]
        ins = []
        for h in range(2):
            u_v, i_v = bufs[h]
            hb = base + h * half
            ins.append(pltpu.async_copy(gut_hbm.at[:, pl.ds(hb, half)], u_v,
                                        sem_in))
            ins.append(pltpu.async_copy(git_hbm.at[:, pl.ds(hb, half)], i_v,
                                        sem_in))

        outs = []
        for h in range(2):
            u_v, i_v = bufs[h]
            hb = base + h * half
            ins[2 * h].wait()
            ins[2 * h + 1].wait()
            outs.append(pltpu.async_copy(u_v, gout_u.at[:, pl.ds(hb, half)],
                                         sem_out))
            outs.append(pltpu.async_copy(i_v, gout_i.at[:, pl.ds(hb, half)],
                                         sem_out))

            def group_body(g, carry, u_v=u_v, i_v=i_v, h=h):
                col = g * _L
                accs = [None, None, None, None]
                for j in range(D):
                    p = u_v[j, pl.ds(col, _L)] * i_v[j, pl.ds(col, _L)]
                    k = j % 4
                    accs[k] = p if accs[k] is None else accs[k] + p
                o_v[pl.ds(h * half + col, _L)] = (
                    (accs[0] + accs[1]) + (accs[2] + accs[3]))
                return carry

            lax.fori_loop(0, half // _L, group_body, 0)

        pltpu.sync_copy(o_v, xui_hbm.at[pl.ds(base, _W)])
        for o in outs:
            o.wait()

    return dot_kernel


_dot = _make_kernel()


def kernel(gu, gi):
    xui, gut_out, git_out = _dot(gu.T, gi.T)
    return (xui, gut_out.T, git_out.T)


# speedup vs baseline: 1.0114x; 1.0114x over previous
import functools

import jax
import jax.numpy as jnp
from jax import lax
from jax.experimental import pallas as pl
from jax.experimental.pallas import tpu as pltpu
from jax.experimental.pallas import tpu_sc as plsc

B = 16384
D = 64
_L = 16

_info = plsc.get_sparse_core_info()
_NC, _NS = _info.num_cores, _info.num_subcores
_NW = _NC * _NS
_W = B // _NW
_GROUPS = _W // _L


def _make_kernel():
    mesh = plsc.VectorSubcoreMesh(core_axis_name="c", subcore_axis_name="s")

    @functools.partial(
        pl.kernel,
        mesh=mesh,
        out_type=[
            jax.ShapeDtypeStruct((B,), jnp.float32),
            jax.ShapeDtypeStruct((D, B), jnp.float32),
            jax.ShapeDtypeStruct((D, B), jnp.float32),
        ],
        scratch_types=[
            pltpu.VMEM((D, _W), jnp.float32),
            pltpu.VMEM((D, _W), jnp.float32),
            pltpu.VMEM((_W,), jnp.float32),
            pltpu.SemaphoreType.DMA,
            pltpu.SemaphoreType.DMA,
        ],
        compiler_params=pltpu.CompilerParams(needs_layout_passes=False),
    )
    def dot_kernel(gut_hbm, git_hbm, xui_hbm, gout_u, gout_i, u_v, i_v, o_v,
                   sem_in, sem_out):
        wid = lax.axis_index("s") * _NC + lax.axis_index("c")
        base = wid * _W
        cu = pltpu.async_copy(gut_hbm.at[:, pl.ds(base, _W)], u_v, sem_in)
        ci = pltpu.async_copy(git_hbm.at[:, pl.ds(base, _W)], i_v, sem_in)
        cu.wait()
        ci.wait()
        ou = pltpu.async_copy(u_v, gout_u.at[:, pl.ds(base, _W)], sem_out)
        oi = pltpu.async_copy(i_v, gout_i.at[:, pl.ds(base, _W)], sem_out)

        def group_body(g, carry):
            col = g * _L
            accs = [None, None, None, None]
            for j in range(D):
                p = u_v[j, pl.ds(col, _L)] * i_v[j, pl.ds(col, _L)]
                k = j % 4
                accs[k] = p if accs[k] is None else accs[k] + p
            o_v[pl.ds(col, _L)] = (accs[0] + accs[1]) + (accs[2] + accs[3])
            return carry

        lax.fori_loop(0, _GROUPS, group_body, 0)
        pltpu.sync_copy(o_v, xui_hbm.at[pl.ds(base, _W)])
        ou.wait()
        oi.wait()

    return dot_kernel


_dot = _make_kernel()


def kernel(gu, gi):
    xui, gut_out, git_out = _dot(gu.T, gi.T)
    return (xui, gut_out.T, git_out.T)
